# Initial kernel scaffold; baseline (speedup 1.0000x reference)
import jax
import jax.numpy as jnp
from jax.experimental import pallas as pl


def kernel(x, W1, b1, W2, b2):
    raise NotImplementedError("write your pallas kernel here")


# speedup vs baseline: 1.4651x; 1.4651x over previous
import functools

import jax
import jax.numpy as jnp
from jax.experimental import pallas as pl
from jax.experimental.pallas import tpu as pltpu


def _router_block_kernel(x_ref, w1_ref, b1_ref, w2_ref, b2_ref, out_ref):
    h = jnp.dot(x_ref[:], w1_ref[:], precision=jax.lax.Precision.DEFAULT)
    h = jnp.maximum(h + b1_ref[:], 0.0)
    logits = jnp.dot(h, w2_ref[:], precision=jax.lax.Precision.DEFAULT)
    logits = logits + b2_ref[:]

    z = logits - jnp.max(logits, axis=1, keepdims=True)
    e = jnp.exp(z)
    p = e / jnp.sum(e, axis=1, keepdims=True)

    n_exp = p.shape[1]
    col = jax.lax.broadcasted_iota(jnp.int32, p.shape, 1)
    m1 = jnp.max(p, axis=1, keepdims=True)
    i1 = jnp.min(jnp.where(p >= m1, col, n_exp), axis=1, keepdims=True)
    p_rest = jnp.where(col == i1, -jnp.inf, p)
    m2 = jnp.max(p_rest, axis=1, keepdims=True)
    i2 = jnp.min(jnp.where(p_rest >= m2, col, n_exp), axis=1, keepdims=True)

    mask = (col == i1) | (col == i2)
    out_ref[:] = jnp.where(mask, p, 0.0) / (m1 + m2 + 1e-8)


@functools.partial(jax.jit, static_argnames=())
def kernel(x, W1, b1, W2, b2):
    n_tokens, d_in = x.shape
    d_hidden = W1.shape[1]
    n_experts = W2.shape[1]
    bm = 512
    grid = (n_tokens // bm,)

    b1_2d = b1.reshape(1, d_hidden)
    b2_2d = b2.reshape(1, n_experts)

    return pl.pallas_call(
        _router_block_kernel,
        grid=grid,
        in_specs=[
            pl.BlockSpec((bm, d_in), lambda i: (i, 0)),
            pl.BlockSpec((d_in, d_hidden), lambda i: (0, 0)),
            pl.BlockSpec((1, d_hidden), lambda i: (0, 0)),
            pl.BlockSpec((d_hidden, n_experts), lambda i: (0, 0)),
            pl.BlockSpec((1, n_experts), lambda i: (0, 0)),
        ],
        out_specs=pl.BlockSpec((bm, n_experts), lambda i: (i, 0)),
        out_shape=jax.ShapeDtypeStruct((n_tokens, n_experts), jnp.float32),
        compiler_params=pltpu.CompilerParams(
            dimension_semantics=("parallel",),
        ),
    )(x, W1, b1_2d, W2, b2_2d)
